# Initial kernel scaffold; baseline (speedup 1.0000x reference)
import jax
import jax.numpy as jnp
from jax.experimental import pallas as pl


def kernel(x, edge_index, W1_0, b1_0, W2_0, b2_0, W1_1, b1_1, W2_1, b2_1, W1_2, b1_2, W2_2, b2_2, bn_gamma, bn_beta, Wo, bo):
    raise NotImplementedError("write your pallas kernel here")


# speedup vs baseline: 2.1121x; 2.1121x over previous
import jax
import jax.numpy as jnp
from jax import lax
from jax.experimental import pallas as pl
from jax.experimental.pallas import tpu as pltpu
from jax.experimental.pallas import tpu_sc as plsc

N = 10000
D = 256
E = 160000

NC = 2
NS = 16
NW = NC * NS
EPW = 5120
EPAD = NW * EPW
CHUNK = 128
NCHB = EPW // CHUNK
NACC = N + 248
DUMP = N


def _segsum_body(x_hbm, srcp_hbm, dstp_hbm, acc_hbm,
                 srcv, dstv, srcc, dstc, rows, sem):
    c = lax.axis_index("c")
    s = lax.axis_index("s")
    wid = s * NC + c
    ebase = wid * EPW
    pltpu.sync_copy(srcp_hbm.at[pl.ds(ebase, EPW)], srcv)
    pltpu.sync_copy(dstp_hbm.at[pl.ds(ebase, EPW)], dstv)

    def chunk_body(j, carry):
        off = j * CHUNK
        for i in range(CHUNK // 16):
            sl = pl.ds(off + i * 16, 16)
            srcc[pl.ds(i * 16, 16)] = srcv[sl]
            dstc[pl.ds(i * 16, 16)] = dstv[sl]
        pltpu.async_copy(x_hbm.at[srcc], rows, sem).wait()
        pltpu.sync_copy(rows, acc_hbm.at[dstc], add=True)
        return carry

    lax.fori_loop(0, NCHB, chunk_body, 0)


_segsum = pl.kernel(
    _segsum_body,
    out_type=(),
    mesh=plsc.VectorSubcoreMesh(core_axis_name="c", subcore_axis_name="s"),
    scratch_types=[
        pltpu.VMEM((EPW,), jnp.int32),
        pltpu.VMEM((EPW,), jnp.int32),
        pltpu.VMEM((CHUNK,), jnp.int32),
        pltpu.VMEM((CHUNK,), jnp.int32),
        pltpu.VMEM((CHUNK, D), jnp.float32),
        pltpu.SemaphoreType.DMA,
    ],
)


BM = 400


def _mlp_body(x_ref, agg_ref, w1_ref, b1_ref, w2_ref, b2_ref, o_ref):
    h = x_ref[...] + agg_ref[...]
    t = jnp.dot(h, w1_ref[...], preferred_element_type=jnp.float32)
    t = jnp.maximum(t + b1_ref[...], 0.0)
    o = jnp.dot(t, w2_ref[...], preferred_element_type=jnp.float32)
    o_ref[...] = jnp.maximum(o + b2_ref[...], 0.0)


def _mlp(x, agg, w1, b1, w2, b2):
    return pl.pallas_call(
        _mlp_body,
        grid=(N // BM,),
        in_specs=[
            pl.BlockSpec((BM, D), lambda b: (b, 0)),
            pl.BlockSpec((BM, D), lambda b: (b, 0)),
            pl.BlockSpec((D, D), lambda b: (0, 0)),
            pl.BlockSpec((1, D), lambda b: (0, 0)),
            pl.BlockSpec((D, D), lambda b: (0, 0)),
            pl.BlockSpec((1, D), lambda b: (0, 0)),
        ],
        out_specs=pl.BlockSpec((BM, D), lambda b: (b, 0)),
        out_shape=jax.ShapeDtypeStruct((N, D), jnp.float32),
    )(x, agg, w1, b1, w2, b2)


def _stats_body(x_ref, sum_ref, sq_ref):
    @pl.when(pl.program_id(0) == 0)
    def _():
        sum_ref[...] = jnp.zeros_like(sum_ref)
        sq_ref[...] = jnp.zeros_like(sq_ref)

    xb = x_ref[...]
    sum_ref[...] += jnp.sum(xb, axis=0, keepdims=True)
    sq_ref[...] += jnp.sum(xb * xb, axis=0, keepdims=True)


def _stats(x):
    return pl.pallas_call(
        _stats_body,
        grid=(N // BM,),
        in_specs=[pl.BlockSpec((BM, D), lambda b: (b, 0))],
        out_specs=[
            pl.BlockSpec((1, D), lambda b: (0, 0)),
            pl.BlockSpec((1, D), lambda b: (0, 0)),
        ],
        out_shape=[
            jax.ShapeDtypeStruct((1, D), jnp.float32),
            jax.ShapeDtypeStruct((1, D), jnp.float32),
        ],
    )(x)


def _bn_proj_body(x_ref, sum_ref, sq_ref, g_ref, bt_ref, wo_ref, bo_ref, o_ref):
    mean = sum_ref[...] * (1.0 / N)
    var = sq_ref[...] * (1.0 / N) - mean * mean
    inv = lax.rsqrt(var + 1e-5)
    xn = (x_ref[...] - mean) * (inv * g_ref[...]) + bt_ref[...]
    o = jnp.dot(xn, wo_ref[...], preferred_element_type=jnp.float32)
    o_ref[...] = o + bo_ref[...]


def _bn_proj(x, sums, sq, gamma, beta, wo, bo):
    return pl.pallas_call(
        _bn_proj_body,
        grid=(N // BM,),
        in_specs=[
            pl.BlockSpec((BM, D), lambda b: (b, 0)),
            pl.BlockSpec((1, D), lambda b: (0, 0)),
            pl.BlockSpec((1, D), lambda b: (0, 0)),
            pl.BlockSpec((1, D), lambda b: (0, 0)),
            pl.BlockSpec((1, D), lambda b: (0, 0)),
            pl.BlockSpec((D, D), lambda b: (0, 0)),
            pl.BlockSpec((1, D), lambda b: (0, 0)),
        ],
        out_specs=pl.BlockSpec((BM, D), lambda b: (b, 0)),
        out_shape=jax.ShapeDtypeStruct((N, D), jnp.float32),
    )(x, sums, sq, gamma, beta, wo, bo)


def kernel(x, edge_index, W1_0, b1_0, W2_0, b2_0, W1_1, b1_1, W2_1, b2_1,
           W1_2, b1_2, W2_2, b2_2, bn_gamma, bn_beta, Wo, bo):
    pad = EPAD - E
    srcp = jnp.concatenate([edge_index[0], jnp.zeros((pad,), jnp.int32)])
    dstp = jnp.concatenate([edge_index[1], jnp.full((pad,), DUMP, jnp.int32)])
    params = [
        (W1_0, b1_0.reshape(1, D), W2_0, b2_0.reshape(1, D)),
        (W1_1, b1_1.reshape(1, D), W2_1, b2_1.reshape(1, D)),
        (W1_2, b1_2.reshape(1, D), W2_2, b2_2.reshape(1, D)),
    ]
    for (w1, b1, w2, b2) in params:
        acc = jax.new_ref(jnp.zeros((NACC, D), jnp.float32))
        _segsum(x, srcp, dstp, acc)
        x = _mlp(x, acc[...], w1, b1, w2, b2)
    sums, sq = _stats(x)
    return _bn_proj(x, sums, sq, bn_gamma.reshape(1, D), bn_beta.reshape(1, D),
                    Wo, bo.reshape(1, D))
